# baseline (speedup 1.0000x reference)
import functools

import jax
import jax.numpy as jnp
from jax import lax
from jax.experimental import pallas as pl
from jax.experimental.pallas import tpu as pltpu
from jax.experimental.pallas import tpu_sc as plsc

N_A = 5000
N_P = 5000
E_TOT = 320000
D = 128

NPAD = 5120
NT = 16
WIN = 100
EPT = E_TOT // NT
NWIN = EPT // WIN
CHUNKS = list(range(0, WIN - 15, 16)) + ([WIN - 16] if WIN % 16 else [])

DEG_ROWS = NPAD
DSTRIPE = DEG_ROWS // NT

ACH = 256
NPASS = (NPAD // 2) // ACH
ACELLS = (ACH + 1) * NPAD

_f32 = jnp.float32
_i32 = jnp.int32
_bf16 = jnp.bfloat16

_mesh = plsc.VectorSubcoreMesh(core_axis_name="c", subcore_axis_name="s")


@functools.partial(
    pl.kernel,
    out_type=jax.ShapeDtypeStruct((2 * DEG_ROWS,), _f32),
    mesh=_mesh,
    scratch_types=[
        pltpu.VMEM((NWIN, WIN), _i32),
        pltpu.VMEM((4, WIN), _i32),
        pltpu.VMEM((WIN,), _f32),
        pltpu.VMEM((DSTRIPE,), _f32),
        pltpu.VMEM_SHARED((DEG_ROWS,), _f32),
        pltpu.SemaphoreType.DMA,
        pltpu.SemaphoreType.DMA,
        pltpu.SemaphoreType.DMA,
        pltpu.SemaphoreType.DMA,
    ],
)
def _sc_degrees(eb_ad, degs, comb_v, sidx_v, ones_v, z_v, deg_sh,
                ds0, ds1, ds2, ds3):
    cid = lax.axis_index("c")
    sid = lax.axis_index("s")

    for c in CHUNKS:
        ones_v[pl.ds(c, 16)] = jnp.ones((16,), _f32)
    for k in range(DSTRIPE // 16):
        z_v[pl.ds(k * 16, 16)] = jnp.zeros((16,), _f32)

    base = sid * DSTRIPE
    pltpu.sync_copy(z_v, deg_sh.at[pl.ds(base, DSTRIPE)])
    pltpu.sync_copy(eb_ad.at[sid], comb_v)

    plsc.subcore_barrier()

    dsems = (ds0, ds1, ds2, ds3)
    use_hi = cid == 0

    def body(g, carry):
        for b in range(4):
            w = 4 * g + b

            @pl.when(g > 0)
            def _():
                pltpu.make_async_copy(ones_v, deg_sh.at[sidx_v.at[b]],
                                      dsems[b]).wait()

            for c in CHUNKS:
                v = comb_v[w, pl.ds(c, 16)]
                hi = lax.shift_right_logical(v, 16)
                lo = lax.bitwise_and(v, jnp.int32(0xFFFF))
                sidx_v[b, pl.ds(c, 16)] = jnp.where(use_hi, hi, lo)
            pltpu.async_copy(ones_v, deg_sh.at[sidx_v.at[b]], dsems[b],
                             add=True)
        return carry

    lax.fori_loop(0, NWIN // 4, body, 0)
    for b in range(4):
        pltpu.make_async_copy(ones_v, deg_sh.at[sidx_v.at[b]], dsems[b]).wait()

    plsc.subcore_barrier()

    pltpu.sync_copy(deg_sh.at[pl.ds(base, DSTRIPE)], z_v)
    pltpu.sync_copy(z_v, degs.at[pl.ds(cid * DEG_ROWS + base, DSTRIPE)])


@functools.partial(
    pl.kernel,
    out_type=jax.ShapeDtypeStruct((NPAD * NPAD,), _f32),
    mesh=_mesh,
    scratch_types=[
        pltpu.VMEM((NWIN, WIN), _i32),
        pltpu.VMEM((4, WIN), _i32),
        pltpu.VMEM((WIN,), _f32),
        pltpu.VMEM((4 * NPAD,), _f32),
        pltpu.VMEM_SHARED((ACELLS,), _f32),
        pltpu.SemaphoreType.DMA,
        pltpu.SemaphoreType.DMA,
        pltpu.SemaphoreType.DMA,
        pltpu.SemaphoreType.DMA,
    ],
)
def _sc_build_a(eb_ad, a_out, comb_v, fidx, ones_v, zrow, chunk,
                cs0, cs1, cs2, cs3):
    cid = lax.axis_index("c")
    sid = lax.axis_index("s")

    for c in CHUNKS:
        ones_v[pl.ds(c, 16)] = jnp.ones((16,), _f32)

    def zbody(i, carry):
        zrow[pl.ds(16 * i, 16)] = jnp.zeros((16,), _f32)
        return carry

    lax.fori_loop(0, 4 * NPAD // 16, zbody, 0)

    pltpu.sync_copy(eb_ad.at[sid], comb_v)

    csems = (cs0, cs1, cs2, cs3)
    rows_per_tile = ACH // NT

    def pass_body(p, carry0):
        gbase = cid * (NPAD // 2) + p * ACH
        for k in range(4):
            pltpu.sync_copy(
                zrow, chunk.at[pl.ds((sid * rows_per_tile + 4 * k) * NPAD,
                                     4 * NPAD)])
        plsc.subcore_barrier()

        def wbody(g, carry):
            for b in range(4):
                w = 4 * g + b

                @pl.when(g > 0)
                def _():
                    pltpu.make_async_copy(ones_v, chunk.at[fidx.at[b]],
                                          csems[b]).wait()

                for c in CHUNKS:
                    v = comb_v[w, pl.ds(c, 16)]
                    srcv = lax.bitwise_and(v, jnp.int32(0xFFFF))
                    dstv = lax.shift_right_logical(v, 16)
                    rel = dstv - gbase
                    ok = jnp.logical_and(rel >= 0, rel < ACH)
                    rel = jnp.where(ok, rel, ACH)
                    fidx[b, pl.ds(c, 16)] = rel * NPAD + srcv
                pltpu.async_copy(ones_v, chunk.at[fidx.at[b]], csems[b],
                                 add=True)
            return carry

        lax.fori_loop(0, NWIN // 4, wbody, 0)
        for b in range(4):
            pltpu.make_async_copy(ones_v, chunk.at[fidx.at[b]],
                                  csems[b]).wait()
        plsc.subcore_barrier()

        pltpu.sync_copy(
            chunk.at[pl.ds(sid * rows_per_tile * NPAD, rows_per_tile * NPAD)],
            a_out.at[pl.ds((gbase + sid * rows_per_tile) * NPAD,
                           rows_per_tile * NPAD)])
        plsc.subcore_barrier()
        return carry0

    lax.fori_loop(0, NPASS, pass_body, 0)


_GRID = NPAD // ACH


def _rs(d_ref):
    return lax.rsqrt(jnp.maximum(d_ref[...], 1.0))


def _agg_step(a_ref, ya, yp, aggp_s, agga_s):
    i = pl.program_id(0)
    ab = a_ref[...].astype(_f32)
    aggp_s[pl.ds(i * ACH, ACH), :] = jnp.dot(ab, ya,
                                             preferred_element_type=_f32)
    contrib = lax.dot_general(ab, yp[pl.ds(i * ACH, ACH), :],
                              (((0,), (0,)), ((), ())),
                              preferred_element_type=_f32)

    @pl.when(i == 0)
    def _():
        agga_s[...] = contrib

    @pl.when(i > 0)
    def _():
        agga_s[...] = agga_s[...] + contrib


def _mid_epilogue(aggp_s, agga_s, da_ref, dp_ref, ba_ref, wa_ref,
                  bp_ref, wp_ref, yo_ref):
    rs_a = _rs(da_ref)
    rs_p = _rs(dp_ref)
    h_a = jax.nn.relu(agga_s[...] * rs_a + ba_ref[...])
    h_p = jax.nn.relu(aggp_s[...] * rs_p + bp_ref[...])
    yo_ref[0] = jnp.dot(h_a * rs_a, wa_ref[...], preferred_element_type=_f32)
    yo_ref[1] = jnp.dot(h_p * rs_p, wp_ref[...], preferred_element_type=_f32)


def _tc_l1_body(a_ref, xa_ref, xp_ref, da_ref, dp_ref, w1a_ref, w1p_ref,
                ba_ref, wa_ref, bp_ref, wp_ref, yo_ref,
                y_s, aggp_s, agga_s):
    @pl.when(pl.program_id(0) == 0)
    def _():
        y_s[0] = jnp.dot(xa_ref[...] * _rs(da_ref), w1a_ref[...],
                         preferred_element_type=_f32)
        y_s[1] = jnp.dot(xp_ref[...] * _rs(dp_ref), w1p_ref[...],
                         preferred_element_type=_f32)

    _agg_step(a_ref, y_s[0], y_s.at[1], aggp_s, agga_s)

    @pl.when(pl.program_id(0) == _GRID - 1)
    def _():
        _mid_epilogue(aggp_s, agga_s, da_ref, dp_ref, ba_ref, wa_ref,
                      bp_ref, wp_ref, yo_ref)


def _tc_l2_body(a_ref, y_ref, da_ref, dp_ref, ba_ref, wa_ref,
                bp_ref, wp_ref, yo_ref, aggp_s, agga_s):
    _agg_step(a_ref, y_ref[0], y_ref.at[1], aggp_s, agga_s)

    @pl.when(pl.program_id(0) == _GRID - 1)
    def _():
        _mid_epilogue(aggp_s, agga_s, da_ref, dp_ref, ba_ref, wa_ref,
                      bp_ref, wp_ref, yo_ref)


def _tc_l3_body(a_ref, y_ref, da_ref, dp_ref, ba_ref, bp_ref,
                oa_ref, op_ref, aggp_s, agga_s):
    _agg_step(a_ref, y_ref[0], y_ref.at[1], aggp_s, agga_s)

    @pl.when(pl.program_id(0) == _GRID - 1)
    def _():
        oa_ref[...] = agga_s[...] * _rs(da_ref) + ba_ref[...]
        op_ref[...] = aggp_s[...] * _rs(dp_ref) + bp_ref[...]


def _full(shape):
    nd = len(shape)
    return pl.BlockSpec(shape, lambda i, _n=nd: (0,) * _n)


_A_SPEC = pl.BlockSpec((ACH, NPAD), lambda i: (i, 0))
_Y_SPEC = _full((2, NPAD, D))
_D_SPEC = _full((NPAD, 1))
_B_SPEC = _full((1, D))
_W_SPEC = _full((D, D))
_X_SPEC = _full((NPAD, D))

_tc_l1 = pl.pallas_call(
    _tc_l1_body,
    grid=(_GRID,),
    in_specs=[_A_SPEC, _X_SPEC, _X_SPEC, _D_SPEC, _D_SPEC, _W_SPEC, _W_SPEC,
              _B_SPEC, _W_SPEC, _B_SPEC, _W_SPEC],
    out_specs=_Y_SPEC,
    out_shape=jax.ShapeDtypeStruct((2, NPAD, D), _f32),
    scratch_shapes=[pltpu.VMEM((2, NPAD, D), _f32),
                    pltpu.VMEM((NPAD, D), _f32),
                    pltpu.VMEM((NPAD, D), _f32)],
)

_tc_l2 = pl.pallas_call(
    _tc_l2_body,
    grid=(_GRID,),
    in_specs=[_A_SPEC, _Y_SPEC, _D_SPEC, _D_SPEC,
              _B_SPEC, _W_SPEC, _B_SPEC, _W_SPEC],
    out_specs=_Y_SPEC,
    out_shape=jax.ShapeDtypeStruct((2, NPAD, D), _f32),
    scratch_shapes=[pltpu.VMEM((NPAD, D), _f32),
                    pltpu.VMEM((NPAD, D), _f32)],
)

_tc_l3 = pl.pallas_call(
    _tc_l3_body,
    grid=(_GRID,),
    in_specs=[_A_SPEC, _Y_SPEC, _D_SPEC, _D_SPEC, _B_SPEC, _B_SPEC],
    out_specs=[_full((NPAD, D)), _full((NPAD, D))],
    out_shape=(jax.ShapeDtypeStruct((NPAD, D), _f32),
               jax.ShapeDtypeStruct((NPAD, D), _f32)),
    scratch_shapes=[pltpu.VMEM((NPAD, D), _f32),
                    pltpu.VMEM((NPAD, D), _f32)],
)


def kernel(x_author, x_paper, edge_src, edge_dst,
           W1w, b1w, W1b, b1b, W2w, b2w, W2b, b2b, W3w, b3w, W3b, b3b):
    src = edge_src.astype(_i32).reshape(NT, NWIN, WIN)
    dst = edge_dst.astype(_i32).reshape(NT, NWIN, WIN)
    eb_ad = src + (dst << 16)

    degs = _sc_degrees(eb_ad)
    deg_p = degs[:DEG_ROWS].reshape(NPAD, 1)
    deg_a = degs[DEG_ROWS:].reshape(NPAD, 1)

    a_mat = _sc_build_a(eb_ad).reshape(NPAD, NPAD)

    xa = jnp.pad(x_author, ((0, NPAD - N_A), (0, 0)))
    xp = jnp.pad(x_paper, ((0, NPAD - N_P), (0, 0)))

    b1w_ = b1w.reshape(1, D)
    b1b_ = b1b.reshape(1, D)
    b2w_ = b2w.reshape(1, D)
    b2b_ = b2b.reshape(1, D)
    b3w_ = b3w.reshape(1, D)
    b3b_ = b3b.reshape(1, D)

    y2 = _tc_l1(a_mat, xa, xp, deg_a, deg_p, W1w, W1b,
                b1b_, W2w, b1w_, W2b)
    y3 = _tc_l2(a_mat, y2, deg_a, deg_p, b2b_, W3w, b2w_, W3b)
    out_a, out_p = _tc_l3(a_mat, y3, deg_a, deg_p, b3b_, b3w_)
    return (out_a[:N_A], out_p[:N_P])


# speedup vs baseline: 1.0053x; 1.0053x over previous
import functools

import jax
import jax.numpy as jnp
from jax import lax
from jax.experimental import pallas as pl
from jax.experimental.pallas import tpu as pltpu
from jax.experimental.pallas import tpu_sc as plsc

N_A = 5000
N_P = 5000
E_TOT = 320000
D = 128

NPAD = 5120
NT = 16
WIN = 100
EPT = E_TOT // NT
NWIN = EPT // WIN
CHUNKS = list(range(0, WIN - 15, 16)) + ([WIN - 16] if WIN % 16 else [])

DEG_ROWS = NPAD
DSTRIPE = DEG_ROWS // NT

ACH = 256
NPASS = (NPAD // 2) // ACH
ACELLS = (ACH + 1) * NPAD

_f32 = jnp.float32
_i32 = jnp.int32
_bf16 = jnp.bfloat16

_mesh = plsc.VectorSubcoreMesh(core_axis_name="c", subcore_axis_name="s")


@functools.partial(
    pl.kernel,
    out_type=(
        jax.ShapeDtypeStruct((2 * DEG_ROWS,), _f32),
        jax.ShapeDtypeStruct((NPAD * NPAD,), _f32),
    ),
    mesh=_mesh,
    scratch_types=[
        pltpu.VMEM((NWIN, WIN), _i32),
        pltpu.VMEM((4, WIN), _i32),
        pltpu.VMEM((WIN,), _f32),
        pltpu.VMEM((NPAD,), _f32),
        pltpu.VMEM((DSTRIPE,), _f32),
        pltpu.VMEM_SHARED((ACELLS,), _f32),
        pltpu.VMEM_SHARED((DEG_ROWS,), _f32),
        pltpu.SemaphoreType.DMA,
        pltpu.SemaphoreType.DMA,
        pltpu.SemaphoreType.DMA,
        pltpu.SemaphoreType.DMA,
    ],
)
def _sc_build_a(eb_ad, degs, a_out, comb_v, fidx, ones_v, zrow, z_v, chunk,
                deg_sh, cs0, cs1, cs2, cs3):
    cid = lax.axis_index("c")
    sid = lax.axis_index("s")

    for c in CHUNKS:
        ones_v[pl.ds(c, 16)] = jnp.ones((16,), _f32)
    for k in range(DSTRIPE // 16):
        z_v[pl.ds(k * 16, 16)] = jnp.zeros((16,), _f32)

    def zbody(i, carry):
        zrow[pl.ds(16 * i, 16)] = jnp.zeros((16,), _f32)
        return carry

    lax.fori_loop(0, NPAD // 16, zbody, 0)

    dbase = sid * DSTRIPE
    pltpu.sync_copy(z_v, deg_sh.at[pl.ds(dbase, DSTRIPE)])
    pltpu.sync_copy(eb_ad.at[sid], comb_v)

    csems = (cs0, cs1, cs2, cs3)
    rows_per_tile = ACH // NT

    plsc.subcore_barrier()
    use_hi = cid == 0

    def dbody(g, carry):
        for b in range(4):
            w = 4 * g + b

            @pl.when(g > 0)
            def _():
                pltpu.make_async_copy(ones_v, deg_sh.at[fidx.at[b]],
                                      csems[b]).wait()

            for c in CHUNKS:
                v = comb_v[w, pl.ds(c, 16)]
                hi = lax.shift_right_logical(v, 16)
                lo = lax.bitwise_and(v, jnp.int32(0xFFFF))
                fidx[b, pl.ds(c, 16)] = jnp.where(use_hi, hi, lo)
            pltpu.async_copy(ones_v, deg_sh.at[fidx.at[b]], csems[b],
                             add=True)
        return carry

    lax.fori_loop(0, NWIN // 4, dbody, 0)
    for b in range(4):
        pltpu.make_async_copy(ones_v, deg_sh.at[fidx.at[b]], csems[b]).wait()
    plsc.subcore_barrier()
    pltpu.sync_copy(deg_sh.at[pl.ds(dbase, DSTRIPE)], z_v)
    pltpu.sync_copy(z_v, degs.at[pl.ds(cid * DEG_ROWS + dbase, DSTRIPE)])


    def pass_body(p, carry0):
        gbase = cid * (NPAD // 2) + p * ACH
        for k in range(rows_per_tile):
            pltpu.sync_copy(
                zrow, chunk.at[pl.ds((sid * rows_per_tile + k) * NPAD,
                                     NPAD)])
        plsc.subcore_barrier()

        def wbody(g, carry):
            for b in range(4):
                w = 4 * g + b

                @pl.when(g > 0)
                def _():
                    pltpu.make_async_copy(ones_v, chunk.at[fidx.at[b]],
                                          csems[b]).wait()

                for c in CHUNKS:
                    v = comb_v[w, pl.ds(c, 16)]
                    srcv = lax.bitwise_and(v, jnp.int32(0xFFFF))
                    dstv = lax.shift_right_logical(v, 16)
                    rel = dstv - gbase
                    ok = jnp.logical_and(rel >= 0, rel < ACH)
                    rel = jnp.where(ok, rel, ACH)
                    fidx[b, pl.ds(c, 16)] = rel * NPAD + srcv
                pltpu.async_copy(ones_v, chunk.at[fidx.at[b]], csems[b],
                                 add=True)
            return carry

        lax.fori_loop(0, NWIN // 4, wbody, 0)
        for b in range(4):
            pltpu.make_async_copy(ones_v, chunk.at[fidx.at[b]],
                                  csems[b]).wait()
        plsc.subcore_barrier()

        pltpu.sync_copy(
            chunk.at[pl.ds(sid * rows_per_tile * NPAD, rows_per_tile * NPAD)],
            a_out.at[pl.ds((gbase + sid * rows_per_tile) * NPAD,
                           rows_per_tile * NPAD)])
        plsc.subcore_barrier()
        return carry0

    lax.fori_loop(0, NPASS, pass_body, 0)


_GRID = NPAD // ACH


def _rs(d_ref):
    return lax.rsqrt(jnp.maximum(d_ref[...], 1.0))


def _agg_step(a_ref, ya, yp, aggp_s, agga_s):
    i = pl.program_id(0)
    ab = a_ref[...].astype(_f32)
    aggp_s[pl.ds(i * ACH, ACH), :] = jnp.dot(ab, ya,
                                             preferred_element_type=_f32)
    contrib = lax.dot_general(ab, yp[pl.ds(i * ACH, ACH), :],
                              (((0,), (0,)), ((), ())),
                              preferred_element_type=_f32)

    @pl.when(i == 0)
    def _():
        agga_s[...] = contrib

    @pl.when(i > 0)
    def _():
        agga_s[...] = agga_s[...] + contrib


def _mid_epilogue(aggp_s, agga_s, da_ref, dp_ref, ba_ref, wa_ref,
                  bp_ref, wp_ref, yo_ref):
    rs_a = _rs(da_ref)
    rs_p = _rs(dp_ref)
    h_a = jax.nn.relu(agga_s[...] * rs_a + ba_ref[...])
    h_p = jax.nn.relu(aggp_s[...] * rs_p + bp_ref[...])
    yo_ref[0] = jnp.dot(h_a * rs_a, wa_ref[...], preferred_element_type=_f32)
    yo_ref[1] = jnp.dot(h_p * rs_p, wp_ref[...], preferred_element_type=_f32)


def _tc_l1_body(a_ref, xa_ref, xp_ref, da_ref, dp_ref, w1a_ref, w1p_ref,
                ba_ref, wa_ref, bp_ref, wp_ref, yo_ref,
                y_s, aggp_s, agga_s):
    @pl.when(pl.program_id(0) == 0)
    def _():
        y_s[0] = jnp.dot(xa_ref[...] * _rs(da_ref), w1a_ref[...],
                         preferred_element_type=_f32)
        y_s[1] = jnp.dot(xp_ref[...] * _rs(dp_ref), w1p_ref[...],
                         preferred_element_type=_f32)

    _agg_step(a_ref, y_s[0], y_s.at[1], aggp_s, agga_s)

    @pl.when(pl.program_id(0) == _GRID - 1)
    def _():
        _mid_epilogue(aggp_s, agga_s, da_ref, dp_ref, ba_ref, wa_ref,
                      bp_ref, wp_ref, yo_ref)


def _tc_l2_body(a_ref, y_ref, da_ref, dp_ref, ba_ref, wa_ref,
                bp_ref, wp_ref, yo_ref, aggp_s, agga_s):
    _agg_step(a_ref, y_ref[0], y_ref.at[1], aggp_s, agga_s)

    @pl.when(pl.program_id(0) == _GRID - 1)
    def _():
        _mid_epilogue(aggp_s, agga_s, da_ref, dp_ref, ba_ref, wa_ref,
                      bp_ref, wp_ref, yo_ref)


def _tc_l3_body(a_ref, y_ref, da_ref, dp_ref, ba_ref, bp_ref,
                oa_ref, op_ref, aggp_s, agga_s):
    _agg_step(a_ref, y_ref[0], y_ref.at[1], aggp_s, agga_s)

    @pl.when(pl.program_id(0) == _GRID - 1)
    def _():
        oa_ref[...] = agga_s[...] * _rs(da_ref) + ba_ref[...]
        op_ref[...] = aggp_s[...] * _rs(dp_ref) + bp_ref[...]


def _full(shape):
    nd = len(shape)
    return pl.BlockSpec(shape, lambda i, _n=nd: (0,) * _n)


_A_SPEC = pl.BlockSpec((ACH, NPAD), lambda i: (i, 0))
_Y_SPEC = _full((2, NPAD, D))
_D_SPEC = _full((NPAD, 1))
_B_SPEC = _full((1, D))
_W_SPEC = _full((D, D))
_X_SPEC = _full((NPAD, D))

_tc_l1 = pl.pallas_call(
    _tc_l1_body,
    grid=(_GRID,),
    in_specs=[_A_SPEC, _X_SPEC, _X_SPEC, _D_SPEC, _D_SPEC, _W_SPEC, _W_SPEC,
              _B_SPEC, _W_SPEC, _B_SPEC, _W_SPEC],
    out_specs=_Y_SPEC,
    out_shape=jax.ShapeDtypeStruct((2, NPAD, D), _f32),
    scratch_shapes=[pltpu.VMEM((2, NPAD, D), _f32),
                    pltpu.VMEM((NPAD, D), _f32),
                    pltpu.VMEM((NPAD, D), _f32)],
)

_tc_l2 = pl.pallas_call(
    _tc_l2_body,
    grid=(_GRID,),
    in_specs=[_A_SPEC, _Y_SPEC, _D_SPEC, _D_SPEC,
              _B_SPEC, _W_SPEC, _B_SPEC, _W_SPEC],
    out_specs=_Y_SPEC,
    out_shape=jax.ShapeDtypeStruct((2, NPAD, D), _f32),
    scratch_shapes=[pltpu.VMEM((NPAD, D), _f32),
                    pltpu.VMEM((NPAD, D), _f32)],
)

_tc_l3 = pl.pallas_call(
    _tc_l3_body,
    grid=(_GRID,),
    in_specs=[_A_SPEC, _Y_SPEC, _D_SPEC, _D_SPEC, _B_SPEC, _B_SPEC],
    out_specs=[_full((NPAD, D)), _full((NPAD, D))],
    out_shape=(jax.ShapeDtypeStruct((NPAD, D), _f32),
               jax.ShapeDtypeStruct((NPAD, D), _f32)),
    scratch_shapes=[pltpu.VMEM((NPAD, D), _f32),
                    pltpu.VMEM((NPAD, D), _f32)],
)


def kernel(x_author, x_paper, edge_src, edge_dst,
           W1w, b1w, W1b, b1b, W2w, b2w, W2b, b2b, W3w, b3w, W3b, b3b):
    src = edge_src.astype(_i32).reshape(NT, NWIN, WIN)
    dst = edge_dst.astype(_i32).reshape(NT, NWIN, WIN)
    eb_ad = src + (dst << 16)

    degs, a_flat = _sc_build_a(eb_ad)
    deg_p = degs[:DEG_ROWS].reshape(NPAD, 1)
    deg_a = degs[DEG_ROWS:].reshape(NPAD, 1)
    a_mat = a_flat.reshape(NPAD, NPAD)

    xa = jnp.pad(x_author, ((0, NPAD - N_A), (0, 0)))
    xp = jnp.pad(x_paper, ((0, NPAD - N_P), (0, 0)))

    b1w_ = b1w.reshape(1, D)
    b1b_ = b1b.reshape(1, D)
    b2w_ = b2w.reshape(1, D)
    b2b_ = b2b.reshape(1, D)
    b3w_ = b3w.reshape(1, D)
    b3b_ = b3b.reshape(1, D)

    y2 = _tc_l1(a_mat, xa, xp, deg_a, deg_p, W1w, W1b,
                b1b_, W2w, b1w_, W2b)
    y3 = _tc_l2(a_mat, y2, deg_a, deg_p, b2b_, W3w, b2w_, W3b)
    out_a, out_p = _tc_l3(a_mat, y3, deg_a, deg_p, b3b_, b3w_)
    return (out_a[:N_A], out_p[:N_P])
